# Initial kernel scaffold; baseline (speedup 1.0000x reference)
import jax
import jax.numpy as jnp
from jax.experimental import pallas as pl


def kernel(token_ids, embeddings, token_weights, W, b):
    raise NotImplementedError("write your pallas kernel here")


# speedup vs baseline: 21.5191x; 21.5191x over previous
import functools

import jax
import jax.numpy as jnp
from jax import lax
from jax.experimental import pallas as pl
from jax.experimental.pallas import tpu as pltpu
from jax.experimental.pallas import tpu_sc as plsc

VOCAB = 100000
HIDDEN = 64
N_LANG = 100
BATCH = 4096
SEQ = 200

LANE = 16
NC, NS = 2, 16
NW = NC * NS
DP = 128

VBLK = 2000
RPW = BATCH // NW
HALF = SEQ // 2


def _scores_tc(embeddings, token_weights, w_t, b_pad):
    def body(emb_ref, tw_ref, wt_ref, b_ref, out_ref):
        x = emb_ref[...] * tw_ref[...]
        out_ref[...] = (
            jnp.dot(x, wt_ref[...], preferred_element_type=jnp.float32)
            + b_ref[...]
        )

    return pl.pallas_call(
        body,
        grid=(VOCAB // VBLK,),
        in_specs=[
            pl.BlockSpec((VBLK, HIDDEN), lambda i: (i, 0)),
            pl.BlockSpec((VBLK, 1), lambda i: (i, 0)),
            pl.BlockSpec((HIDDEN, DP), lambda i: (0, 0)),
            pl.BlockSpec((1, DP), lambda i: (0, 0)),
        ],
        out_specs=pl.BlockSpec((VBLK, DP), lambda i: (i, 0)),
        out_shape=jax.ShapeDtypeStruct((VOCAB, DP), jnp.float32),
    )(embeddings, token_weights, w_t, b_pad)


def _gather_max_sc(ids2, scores):
    mesh = plsc.VectorSubcoreMesh(core_axis_name="c", subcore_axis_name="s")

    @functools.partial(
        pl.kernel,
        out_type=jax.ShapeDtypeStruct((BATCH, DP), jnp.float32),
        mesh=mesh,
        scratch_types=[
            pltpu.VMEM((2 * RPW, HALF), jnp.int32),
            pltpu.VMEM((SEQ, DP), jnp.float32),
            pltpu.VMEM((RPW, DP), jnp.float32),
            pltpu.SemaphoreType.DMA,
        ],
    )
    def run(ids_hbm, table_hbm, out_hbm, ids_v, rows_v, out_v, sem):
        wid = lax.axis_index("s") * NC + lax.axis_index("c")
        pltpu.sync_copy(ids_hbm.at[pl.ds(wid * 2 * RPW, 2 * RPW)], ids_v)

        def row_body(r, carry):
            pltpu.async_copy(
                table_hbm.at[ids_v.at[2 * r]], rows_v.at[pl.ds(0, HALF)], sem
            ).wait()
            pltpu.async_copy(
                table_hbm.at[ids_v.at[2 * r + 1]],
                rows_v.at[pl.ds(HALF, HALF)],
                sem,
            ).wait()

            def red(j, acc):
                return tuple(
                    jnp.maximum(acc[k], rows_v[j, pl.ds(k * LANE, LANE)])
                    for k in range(DP // LANE)
                )

            acc0 = tuple(
                jnp.full((LANE,), -jnp.inf, jnp.float32)
                for _ in range(DP // LANE)
            )
            acc = lax.fori_loop(0, SEQ, red, acc0)
            for k in range(DP // LANE):
                out_v[r, pl.ds(k * LANE, LANE)] = acc[k]
            return carry

        lax.fori_loop(0, RPW, row_body, 0)
        pltpu.sync_copy(out_v, out_hbm.at[pl.ds(wid * RPW, RPW)])

    return run(ids2, scores)


def kernel(token_ids, embeddings, token_weights, W, b):
    w_t = jnp.zeros((HIDDEN, DP), jnp.float32).at[:, :N_LANG].set(W.T)
    b_pad = jnp.zeros((1, DP), jnp.float32).at[0, :N_LANG].set(b)
    scores = _scores_tc(embeddings, token_weights, w_t, b_pad)
    ids2 = token_ids.reshape(2 * BATCH, HALF)
    out = _gather_max_sc(ids2, scores)
    return out[:, :N_LANG]


# speedup vs baseline: 37.0409x; 1.7213x over previous
import functools

import jax
import jax.numpy as jnp
from jax import lax
from jax.experimental import pallas as pl
from jax.experimental.pallas import tpu as pltpu
from jax.experimental.pallas import tpu_sc as plsc

VOCAB = 100000
HIDDEN = 64
N_LANG = 100
BATCH = 4096
SEQ = 200

LANE = 16
NC, NS = 2, 16
NW = NC * NS
DP = 128
NK = DP // LANE

VBLK = 2000
RPW = BATCH // NW
HALF = SEQ // 2


def _scores_tc(embeddings, token_weights, w_t, b_pad):
    def body(emb_ref, tw_ref, wt_ref, b_ref, out_ref):
        x = emb_ref[...] * tw_ref[...]
        out_ref[...] = (
            jnp.dot(x, wt_ref[...], preferred_element_type=jnp.float32)
            + b_ref[...]
        )

    return pl.pallas_call(
        body,
        grid=(VOCAB // VBLK,),
        in_specs=[
            pl.BlockSpec((VBLK, HIDDEN), lambda i: (i, 0)),
            pl.BlockSpec((VBLK, 1), lambda i: (i, 0)),
            pl.BlockSpec((HIDDEN, DP), lambda i: (0, 0)),
            pl.BlockSpec((1, DP), lambda i: (0, 0)),
        ],
        out_specs=pl.BlockSpec((VBLK, DP), lambda i: (i, 0)),
        out_shape=jax.ShapeDtypeStruct((VOCAB, DP), jnp.float32),
    )(embeddings, token_weights, w_t, b_pad)


def _gather_max_sc(ids2, scores):
    mesh = plsc.VectorSubcoreMesh(core_axis_name="c", subcore_axis_name="s")

    @functools.partial(
        pl.kernel,
        out_type=jax.ShapeDtypeStruct((BATCH, DP), jnp.float32),
        mesh=mesh,
        scratch_types=[
            pltpu.VMEM((2 * RPW, HALF), jnp.int32),
            pltpu.VMEM((SEQ, DP), jnp.float32),
            pltpu.VMEM((SEQ, DP), jnp.float32),
            pltpu.VMEM((RPW, DP), jnp.float32),
            pltpu.SemaphoreType.DMA,
            pltpu.SemaphoreType.DMA,
        ],
    )
    def run(ids_hbm, table_hbm, out_hbm, ids_v, buf_a, buf_b, out_v,
            sem_a, sem_b):
        wid = lax.axis_index("s") * NC + lax.axis_index("c")
        pltpu.sync_copy(ids_hbm.at[pl.ds(wid * 2 * RPW, 2 * RPW)], ids_v)

        def fire(r, buf, sem):
            pltpu.async_copy(
                table_hbm.at[ids_v.at[2 * r]], buf.at[pl.ds(0, HALF)], sem
            )
            pltpu.async_copy(
                table_hbm.at[ids_v.at[2 * r + 1]],
                buf.at[pl.ds(HALF, HALF)],
                sem,
            )

        def drain(buf, sem):
            pltpu.make_async_copy(
                table_hbm.at[ids_v.at[0]], buf.at[pl.ds(0, HALF)], sem
            ).wait()
            pltpu.make_async_copy(
                table_hbm.at[ids_v.at[0]], buf.at[pl.ds(HALF, HALF)], sem
            ).wait()

        def reduce_into(r, buf):
            def step(j, acc):
                return tuple(
                    jnp.maximum(acc[k], buf[j, pl.ds(k * LANE, LANE)])
                    for k in range(NK)
                )

            acc0 = tuple(
                jnp.full((LANE,), -jnp.inf, jnp.float32) for _ in range(NK)
            )
            acc = lax.fori_loop(0, SEQ, step, acc0)
            for k in range(NK):
                out_v[r, pl.ds(k * LANE, LANE)] = acc[k]

        fire(0, buf_a, sem_a)

        def pair_body(p, carry):
            r0 = 2 * p
            fire(r0 + 1, buf_b, sem_b)
            drain(buf_a, sem_a)
            reduce_into(r0, buf_a)
            fire(jnp.minimum(r0 + 2, RPW - 1), buf_a, sem_a)
            drain(buf_b, sem_b)
            reduce_into(r0 + 1, buf_b)
            return carry

        lax.fori_loop(0, RPW // 2, pair_body, 0)
        drain(buf_a, sem_a)
        pltpu.sync_copy(out_v, out_hbm.at[pl.ds(wid * RPW, RPW)])

    return run(ids2, scores)


def kernel(token_ids, embeddings, token_weights, W, b):
    w_t = jnp.zeros((HIDDEN, DP), jnp.float32).at[:, :N_LANG].set(W.T)
    b_pad = jnp.zeros((1, DP), jnp.float32).at[0, :N_LANG].set(b)
    scores = _scores_tc(embeddings, token_weights, w_t, b_pad)
    ids2 = token_ids.reshape(2 * BATCH, HALF)
    out = _gather_max_sc(ids2, scores)
    return out[:, :N_LANG]


# speedup vs baseline: 42.4720x; 1.1466x over previous
import functools

import jax
import jax.numpy as jnp
from jax import lax
from jax.experimental import pallas as pl
from jax.experimental.pallas import tpu as pltpu
from jax.experimental.pallas import tpu_sc as plsc

VOCAB = 100000
HIDDEN = 64
N_LANG = 100
BATCH = 4096
SEQ = 200

LANE = 16
NC, NS = 2, 16
NW = NC * NS
DP = 128
NK = DP // LANE

VBLK = 4000
RPW = BATCH // NW
HALF = SEQ // 2


def _scores_tc(embeddings, tw_row, w_t, b_pad):
    def body(emb_ref, tw_ref, wt_ref, b_ref, out_ref):
        y = jnp.dot(
            emb_ref[...], wt_ref[...], preferred_element_type=jnp.float32
        )
        scale = lax.dot_general(
            tw_ref[0],
            jnp.full((1, DP), 1.0, jnp.float32),
            (((0,), (0,)), ((), ())),
            preferred_element_type=jnp.float32,
        )
        out_ref[...] = y * scale + b_ref[...]

    return pl.pallas_call(
        body,
        grid=(VOCAB // VBLK,),
        in_specs=[
            pl.BlockSpec((VBLK, HIDDEN), lambda i: (i, 0)),
            pl.BlockSpec((1, 1, VBLK), lambda i: (i, 0, 0)),
            pl.BlockSpec((HIDDEN, DP), lambda i: (0, 0)),
            pl.BlockSpec((1, DP), lambda i: (0, 0)),
        ],
        out_specs=pl.BlockSpec((VBLK, DP), lambda i: (i, 0)),
        out_shape=jax.ShapeDtypeStruct((VOCAB, DP), jnp.float32),
    )(embeddings, tw_row, w_t, b_pad)


def _gather_max_sc(ids2, scores):
    mesh = plsc.VectorSubcoreMesh(core_axis_name="c", subcore_axis_name="s")

    @functools.partial(
        pl.kernel,
        out_type=jax.ShapeDtypeStruct((BATCH, DP), jnp.float32),
        mesh=mesh,
        scratch_types=[
            pltpu.VMEM((2 * RPW, HALF), jnp.int32),
            pltpu.VMEM((SEQ, DP), jnp.float32),
            pltpu.VMEM((SEQ, DP), jnp.float32),
            pltpu.VMEM((RPW, DP), jnp.float32),
            pltpu.SemaphoreType.DMA,
            pltpu.SemaphoreType.DMA,
        ],
    )
    def run(ids_hbm, table_hbm, out_hbm, ids_v, buf_a, buf_b, out_v,
            sem_a, sem_b):
        wid = lax.axis_index("s") * NC + lax.axis_index("c")
        pltpu.sync_copy(ids_hbm.at[pl.ds(wid * 2 * RPW, 2 * RPW)], ids_v)

        def fire(r, buf, sem):
            pltpu.async_copy(
                table_hbm.at[ids_v.at[2 * r]], buf.at[pl.ds(0, HALF)], sem
            )
            pltpu.async_copy(
                table_hbm.at[ids_v.at[2 * r + 1]],
                buf.at[pl.ds(HALF, HALF)],
                sem,
            )

        def drain(buf, sem):
            pltpu.make_async_copy(
                table_hbm.at[ids_v.at[0]], buf.at[pl.ds(0, HALF)], sem
            ).wait()
            pltpu.make_async_copy(
                table_hbm.at[ids_v.at[0]], buf.at[pl.ds(HALF, HALF)], sem
            ).wait()

        def reduce_into(r, buf):
            def step(j, acc):
                return tuple(
                    jnp.maximum(acc[k], buf[j, pl.ds(k * LANE, LANE)])
                    for k in range(NK)
                )

            acc0 = tuple(
                jnp.full((LANE,), -jnp.inf, jnp.float32) for _ in range(NK)
            )
            acc = lax.fori_loop(0, SEQ, step, acc0)
            for k in range(NK):
                out_v[r, pl.ds(k * LANE, LANE)] = acc[k]

        fire(0, buf_a, sem_a)

        def pair_body(p, carry):
            r0 = 2 * p
            fire(r0 + 1, buf_b, sem_b)
            drain(buf_a, sem_a)
            reduce_into(r0, buf_a)
            fire(jnp.minimum(r0 + 2, RPW - 1), buf_a, sem_a)
            drain(buf_b, sem_b)
            reduce_into(r0 + 1, buf_b)
            return carry

        lax.fori_loop(0, RPW // 2, pair_body, 0)
        drain(buf_a, sem_a)
        pltpu.sync_copy(out_v, out_hbm.at[pl.ds(wid * RPW, RPW)])

    return run(ids2, scores)


def kernel(token_ids, embeddings, token_weights, W, b):
    w_t = jnp.zeros((HIDDEN, DP), jnp.float32).at[:, :N_LANG].set(W.T)
    b_pad = jnp.zeros((1, DP), jnp.float32).at[0, :N_LANG].set(b)
    tw_row = token_weights.reshape(VOCAB // VBLK, 1, VBLK)
    scores = _scores_tc(embeddings, tw_row, w_t, b_pad)
    ids2 = token_ids.reshape(2 * BATCH, HALF)
    out = _gather_max_sc(ids2, scores)
    return out[:, :N_LANG]


# speedup vs baseline: 42.5812x; 1.0026x over previous
import functools

import jax
import jax.numpy as jnp
from jax import lax
from jax.experimental import pallas as pl
from jax.experimental.pallas import tpu as pltpu
from jax.experimental.pallas import tpu_sc as plsc

VOCAB = 100000
HIDDEN = 64
N_LANG = 100
BATCH = 4096
SEQ = 200

LANE = 16
NC, NS = 2, 16
NW = NC * NS
DP = 128
NK = DP // LANE
NKC = 7
UNROLL = 4

VBLK = 4000
RPW = BATCH // NW
HALF = SEQ // 2


def _scores_tc(embeddings, tw_row, w_t, b_pad):
    def body(emb_ref, tw_ref, wt_ref, b_ref, out_ref):
        y = jnp.dot(
            emb_ref[...], wt_ref[...], preferred_element_type=jnp.float32
        )
        scale = lax.dot_general(
            tw_ref[0],
            jnp.full((1, DP), 1.0, jnp.float32),
            (((0,), (0,)), ((), ())),
            preferred_element_type=jnp.float32,
        )
        out_ref[...] = y * scale + b_ref[...]

    return pl.pallas_call(
        body,
        grid=(VOCAB // VBLK,),
        in_specs=[
            pl.BlockSpec((VBLK, HIDDEN), lambda i: (i, 0)),
            pl.BlockSpec((1, 1, VBLK), lambda i: (i, 0, 0)),
            pl.BlockSpec((HIDDEN, DP), lambda i: (0, 0)),
            pl.BlockSpec((1, DP), lambda i: (0, 0)),
        ],
        out_specs=pl.BlockSpec((VBLK, DP), lambda i: (i, 0)),
        out_shape=jax.ShapeDtypeStruct((VOCAB, DP), jnp.float32),
    )(embeddings, tw_row, w_t, b_pad)


def _gather_max_sc(ids2, scores):
    mesh = plsc.VectorSubcoreMesh(core_axis_name="c", subcore_axis_name="s")

    @functools.partial(
        pl.kernel,
        out_type=jax.ShapeDtypeStruct((BATCH, DP), jnp.float32),
        mesh=mesh,
        scratch_types=[
            pltpu.VMEM((2 * RPW, HALF), jnp.int32),
            pltpu.VMEM((SEQ, DP), jnp.float32),
            pltpu.VMEM((SEQ, DP), jnp.float32),
            pltpu.VMEM((RPW, DP), jnp.float32),
            pltpu.SemaphoreType.DMA,
            pltpu.SemaphoreType.DMA,
        ],
    )
    def run(ids_hbm, table_hbm, out_hbm, ids_v, buf_a, buf_b, out_v,
            sem_a, sem_b):
        wid = lax.axis_index("s") * NC + lax.axis_index("c")
        pltpu.sync_copy(ids_hbm.at[pl.ds(wid * 2 * RPW, 2 * RPW)], ids_v)

        def fire(r, buf, sem):
            pltpu.async_copy(
                table_hbm.at[ids_v.at[2 * r]], buf.at[pl.ds(0, HALF)], sem
            )
            pltpu.async_copy(
                table_hbm.at[ids_v.at[2 * r + 1]],
                buf.at[pl.ds(HALF, HALF)],
                sem,
            )

        def drain(buf, sem):
            pltpu.make_async_copy(
                table_hbm.at[ids_v.at[0]], buf.at[pl.ds(0, HALF)], sem
            ).wait()
            pltpu.make_async_copy(
                table_hbm.at[ids_v.at[0]], buf.at[pl.ds(HALF, HALF)], sem
            ).wait()

        def reduce_into(r, buf):
            def step(jj, acc):
                j0 = UNROLL * jj
                for u in range(UNROLL):
                    acc = tuple(
                        jnp.maximum(
                            acc[k], buf[j0 + u, pl.ds(k * LANE, LANE)]
                        )
                        for k in range(NKC)
                    )
                return acc

            acc0 = tuple(
                jnp.full((LANE,), -jnp.inf, jnp.float32) for _ in range(NKC)
            )
            acc = lax.fori_loop(0, SEQ // UNROLL, step, acc0)
            for k in range(NKC):
                out_v[r, pl.ds(k * LANE, LANE)] = acc[k]

        fire(0, buf_a, sem_a)

        def pair_body(p, carry):
            r0 = 2 * p
            fire(r0 + 1, buf_b, sem_b)
            drain(buf_a, sem_a)
            reduce_into(r0, buf_a)
            fire(jnp.minimum(r0 + 2, RPW - 1), buf_a, sem_a)
            drain(buf_b, sem_b)
            reduce_into(r0 + 1, buf_b)
            return carry

        lax.fori_loop(0, RPW // 2, pair_body, 0)
        drain(buf_a, sem_a)
        pltpu.sync_copy(out_v, out_hbm.at[pl.ds(wid * RPW, RPW)])

    return run(ids2, scores)


def kernel(token_ids, embeddings, token_weights, W, b):
    w_t = jnp.zeros((HIDDEN, DP), jnp.float32).at[:, :N_LANG].set(W.T)
    b_pad = jnp.zeros((1, DP), jnp.float32).at[0, :N_LANG].set(b)
    tw_row = token_weights.reshape(VOCAB // VBLK, 1, VBLK)
    scores = _scores_tc(embeddings, tw_row, w_t, b_pad)
    ids2 = token_ids.reshape(2 * BATCH, HALF)
    out = _gather_max_sc(ids2, scores)
    return out[:, :N_LANG]


# speedup vs baseline: 48.3820x; 1.1362x over previous
import functools

import jax
import jax.numpy as jnp
from jax import lax
from jax.experimental import pallas as pl
from jax.experimental.pallas import tpu as pltpu
from jax.experimental.pallas import tpu_sc as plsc

VOCAB = 100000
HIDDEN = 64
N_LANG = 100
BATCH = 4096
SEQ = 200

LANE = 16
NC, NS = 2, 16
NW = NC * NS
DP = 128
NK = DP // LANE
NKC = 7
UNROLL = 4

VBLK = 4000
RPW = BATCH // NW
HALF = SEQ // 2


def _scores_tc(embeddings, tw_row, w_t, b_pad):
    def body(emb_ref, tw_ref, wt_ref, b_ref, out_ref):
        y = jnp.dot(
            emb_ref[...], wt_ref[...], preferred_element_type=jnp.float32
        )
        scale = lax.dot_general(
            tw_ref[0],
            jnp.full((1, DP), 1.0, jnp.float32),
            (((0,), (0,)), ((), ())),
            preferred_element_type=jnp.float32,
        )
        out_ref[...] = y * scale + b_ref[...]

    return pl.pallas_call(
        body,
        grid=(VOCAB // VBLK,),
        in_specs=[
            pl.BlockSpec((VBLK, HIDDEN), lambda i: (i, 0)),
            pl.BlockSpec((1, 1, VBLK), lambda i: (i, 0, 0)),
            pl.BlockSpec((HIDDEN, DP), lambda i: (0, 0)),
            pl.BlockSpec((1, DP), lambda i: (0, 0)),
        ],
        out_specs=pl.BlockSpec((VBLK, DP), lambda i: (i, 0)),
        out_shape=jax.ShapeDtypeStruct((VOCAB, DP), jnp.float32),
    )(embeddings, tw_row, w_t, b_pad)


def _gather_max_sc(ids2, scores):
    mesh = plsc.VectorSubcoreMesh(core_axis_name="c", subcore_axis_name="s")

    @functools.partial(
        pl.kernel,
        out_type=jax.ShapeDtypeStruct((BATCH, DP), jnp.float32),
        mesh=mesh,
        scratch_types=[
            pltpu.VMEM((2 * RPW, HALF), jnp.int32),
            pltpu.VMEM((HALF, DP), jnp.float32),
            pltpu.VMEM((HALF, DP), jnp.float32),
            pltpu.VMEM((HALF, DP), jnp.float32),
            pltpu.VMEM((HALF, DP), jnp.float32),
            pltpu.VMEM((RPW, DP), jnp.float32),
            pltpu.SemaphoreType.DMA,
            pltpu.SemaphoreType.DMA,
            pltpu.SemaphoreType.DMA,
            pltpu.SemaphoreType.DMA,
        ],
    )
    def run(ids_hbm, table_hbm, out_hbm, ids_v, buf_a, buf_b, buf_c, buf_d,
            out_v, sem_a, sem_b, sem_c, sem_d):
        wid = lax.axis_index("s") * NC + lax.axis_index("c")
        pltpu.sync_copy(ids_hbm.at[pl.ds(wid * 2 * RPW, 2 * RPW)], ids_v)
        nunits = 2 * RPW

        def fire(u, buf, sem):
            pltpu.async_copy(
                table_hbm.at[ids_v.at[jnp.minimum(u, nunits - 1)]], buf, sem
            )

        def drain(buf, sem):
            pltpu.make_async_copy(table_hbm.at[ids_v.at[0]], buf, sem).wait()

        def reduce_unit(buf, acc):
            def step(jj, a):
                j0 = UNROLL * jj
                for u in range(UNROLL):
                    a = tuple(
                        jnp.maximum(
                            a[k], buf[j0 + u, pl.ds(k * LANE, LANE)]
                        )
                        for k in range(NKC)
                    )
                return a

            return lax.fori_loop(0, HALF // UNROLL, step, acc)

        def fresh():
            return tuple(
                jnp.full((LANE,), -jnp.inf, jnp.float32) for _ in range(NKC)
            )

        def store(r, acc):
            for k in range(NKC):
                out_v[r, pl.ds(k * LANE, LANE)] = acc[k]

        fire(0, buf_a, sem_a)
        fire(1, buf_b, sem_b)
        fire(2, buf_c, sem_c)

        def quad_body(p, carry):
            u0 = 4 * p
            fire(u0 + 3, buf_d, sem_d)
            drain(buf_a, sem_a)
            acc = reduce_unit(buf_a, fresh())
            fire(u0 + 4, buf_a, sem_a)
            drain(buf_b, sem_b)
            acc = reduce_unit(buf_b, acc)
            store(2 * p, acc)
            fire(u0 + 5, buf_b, sem_b)
            drain(buf_c, sem_c)
            acc = reduce_unit(buf_c, fresh())
            fire(u0 + 6, buf_c, sem_c)
            drain(buf_d, sem_d)
            acc = reduce_unit(buf_d, acc)
            store(2 * p + 1, acc)
            return carry

        lax.fori_loop(0, RPW // 2, quad_body, 0)
        drain(buf_a, sem_a)
        drain(buf_b, sem_b)
        drain(buf_c, sem_c)
        pltpu.sync_copy(out_v, out_hbm.at[pl.ds(wid * RPW, RPW)])

    return run(ids2, scores)


def kernel(token_ids, embeddings, token_weights, W, b):
    w_t = jnp.zeros((HIDDEN, DP), jnp.float32).at[:, :N_LANG].set(W.T)
    b_pad = jnp.zeros((1, DP), jnp.float32).at[0, :N_LANG].set(b)
    tw_row = token_weights.reshape(VOCAB // VBLK, 1, VBLK)
    scores = _scores_tc(embeddings, tw_row, w_t, b_pad)
    ids2 = token_ids.reshape(2 * BATCH, HALF)
    out = _gather_max_sc(ids2, scores)
    return out[:, :N_LANG]
